# baseline (speedup 1.0000x reference)
import functools

import jax
import jax.numpy as jnp
from jax import lax
from jax.experimental import pallas as pl
from jax.experimental.pallas import tpu as pltpu
from jax.experimental.pallas import tpu_sc as plsc

_N = 4096
_MAX_PED = 16
_MIN_R = 0.5
_MAX_R = 10.0
_CAP = 400.0
_BIG = 1.0e9
_NW = 32
_SC_ROWS = 512
_RPW = _SC_ROWS // _NW
_NC = 32
_PAD = 256
_BR = 512


def _iota16():
    return lax.iota(jnp.int32, 16)


def _sc_body(x_hbm, y_hbm, hx_hbm, hy_hbm, s15_hbm, s16_hbm,
             xv, yv, hxv, hyv, rowbuf, foldbuf, idbuf, candbuf,
             out15, out16):
    wid = lax.axis_index("s") * 2 + lax.axis_index("c")
    base = wid * _RPW
    pltpu.sync_copy(x_hbm, xv)
    pltpu.sync_copy(y_hbm, yv)
    pltpu.sync_copy(hx_hbm, hxv)
    pltpu.sync_copy(hy_hbm, hyv)
    cap_v = jnp.full((16,), _CAP, jnp.float32)
    for c in range(_PAD // 16):
        rowbuf[pl.ds(_N + c * 16, 16)] = cap_v

    def do_row(iv, _):
        ri = base + iv
        ri_v = _iota16() * 0 + ri
        xi = plsc.load_gather(xv, [ri_v])
        yi = plsc.load_gather(yv, [ri_v])
        hxi = plsc.load_gather(hxv, [ri_v])
        hyi = plsc.load_gather(hyv, [ri_v])
        sqh = hxi * hxi + hyi * hyi

        def do_group(g, _):
            def do_chunk(c8, acc):
                c = g * 16 + c8
                xj = xv[pl.ds(c * 16, 16)]
                yj = yv[pl.ds(c * 16, 16)]
                dx = xj - xi
                dy = yj - yi
                sq = dx * dx + dy * dy
                dt = dx * hxi + dy * hyi
                ok = (dt > 0.0) & (4.0 * (dt * dt) > sq * sqh) & (sq < _CAP)
                val = jnp.where(ok, sq, _CAP)
                rowbuf[pl.ds(c * 16, 16)] = val
                return jnp.minimum(acc, val)

            acc = lax.fori_loop(0, 16, do_chunk, cap_v)
            foldbuf[pl.ds(g * 16, 16)] = acc
            return 0

        lax.fori_loop(0, 16, do_group, 0)

        def t_round(k, m):
            m_v = _iota16() * 0.0 + m

            def scan_g(g, cur):
                fv = foldbuf[pl.ds(g * 16, 16)]
                return jnp.minimum(cur, jnp.where(fv > m_v, fv, _BIG))

            cur = lax.fori_loop(0, 16, scan_g, jnp.full((16,), _BIG, jnp.float32))
            return jnp.min(cur)

        t = lax.fori_loop(0, _MAX_PED + 1, t_round, jnp.float32(-1.0))
        t_v = _iota16() * 0.0 + t

        dummy = jnp.full((16,), 256, jnp.int32)
        for s in range(_NC // 16):
            idbuf[pl.ds(s * 16, 16)] = dummy
        off = jnp.int32(0)
        for g in range(16):
            fv = foldbuf[pl.ds(g * 16, 16)]
            sel = (fv <= t_v) & (fv < _CAP)
            ids = _iota16() + 16 * g
            plsc.store_compressed(idbuf.at[pl.ds(off, 16)], ids, mask=sel)
            off = off + jnp.sum(sel.astype(jnp.int32))

        for s in range(_NC // 16):
            ids = idbuf[pl.ds(s * 16, 16)]
            gq = lax.shift_right_logical(ids, 4)
            lq = lax.bitwise_and(ids, 15)
            for c in range(16):
                cols = 256 * gq + (16 * c) + lq
                candbuf[pl.ds((s * 16 + c) * 16, 16)] = plsc.load_gather(
                    rowbuf, [cols])

        def sel_round(k, carry):
            prev, s15, s16, cum = carry
            prev_v = _iota16() * 0.0 + prev

            def mscan(i, cur):
                v = candbuf[pl.ds(i * 16, 16)]
                return jnp.minimum(cur, jnp.where(v > prev_v, v, _BIG))

            cur = lax.fori_loop(0, _NC, mscan,
                                jnp.full((16,), _BIG, jnp.float32))
            m = jnp.min(cur)
            m_v = _iota16() * 0.0 + m

            def cscan(i, cv):
                v = candbuf[pl.ds(i * 16, 16)]
                return cv + (v == m_v).astype(jnp.int32)

            cnt_v = lax.fori_loop(0, _NC, cscan, jnp.zeros((16,), jnp.int32))
            cnum = jnp.sum(cnt_v)
            nxt = cum + cnum
            s15 = jnp.where((cum <= _MAX_PED - 1) & (nxt > _MAX_PED - 1), m, s15)
            s16 = jnp.where((cum <= _MAX_PED) & (nxt > _MAX_PED), m, s16)
            return m, s15, s16, nxt

        _, s15, s16, _ = lax.fori_loop(
            0, _MAX_PED + 1, sel_round,
            (jnp.float32(-1.0), jnp.float32(_CAP), jnp.float32(_CAP),
             jnp.int32(0)))

        lane0 = _iota16() == 0
        iv_v = _iota16() * 0 + iv
        plsc.store_scatter(out15, [iv_v], _iota16() * 0.0 + s15, mask=lane0)
        plsc.store_scatter(out16, [iv_v], _iota16() * 0.0 + s16, mask=lane0)
        return 0

    lax.fori_loop(0, _RPW, do_row, 0)
    pltpu.sync_copy(out15, s15_hbm.at[pl.ds(base, _RPW)])
    pltpu.sync_copy(out16, s16_hbm.at[pl.ds(base, _RPW)])


def _tc_body(xrow, yrow, xcol, ycol, hxcol, hycol, s15_ref, s16_ref):
    xi = xcol[...]
    yi = ycol[...]
    hx = hxcol[...]
    hy = hycol[...]
    sqh = hx * hx + hy * hy
    xj = xrow[...]
    yj = yrow[...]
    dx = xj - xi
    dy = yj - yi
    squ = dx * dx + dy * dy
    dot = dx * hx + dy * hy
    in_sight = (dot > 0.0) & (4.0 * (dot * dot) > squ * sqh)
    inf = jnp.float32(jnp.inf)
    msk = jnp.where(in_sight, squ, inf)

    ms = []
    m = jnp.full((_BR, 1), -jnp.inf, jnp.float32)
    for _ in range(_MAX_PED + 1):
        m = jnp.min(jnp.where(msk > m, msk, inf), axis=1, keepdims=True)
        ms.append(m)

    cnt17 = jnp.sum((msk <= ms[_MAX_PED]).astype(jnp.int32), axis=1,
                    keepdims=True)
    ties = jnp.any((ms[_MAX_PED] < inf) & (cnt17 > _MAX_PED + 1))

    def fast(_):
        return ms[_MAX_PED - 1], ms[_MAX_PED]

    def slow(_):
        s15 = jnp.full((_BR, 1), inf)
        s16 = jnp.full((_BR, 1), inf)
        cum = jnp.zeros((_BR, 1), jnp.int32)
        for k in range(_MAX_PED + 1):
            c = jnp.sum((msk == ms[k]).astype(jnp.int32), axis=1,
                        keepdims=True)
            nxt = cum + c
            s15 = jnp.where((cum <= _MAX_PED - 1) & (nxt > _MAX_PED - 1),
                            ms[k], s15)
            s16 = jnp.where((cum <= _MAX_PED) & (nxt > _MAX_PED), ms[k], s16)
            cum = nxt
        return s15, s16

    s15, s16 = jax.lax.cond(ties, slow, fast, None)
    s15_ref[...] = s15
    s16_ref[...] = s16


def _finish_body(s15_ref, s16_ref, idx_ref, rad_ref, out_ref):
    r = 0.5 * (jnp.sqrt(s15_ref[...]) + jnp.sqrt(s16_ref[...]))
    r = jnp.clip(r, _MIN_R, _MAX_R)
    out_ref[...] = jnp.where(idx_ref[...] != 0, r, rad_ref[...])


def kernel(past_ped_positions, ped_positions, indexes, all_radii):
    n = ped_positions.shape[0]
    x = ped_positions[:, 0]
    y = ped_positions[:, 1]
    hx = x - past_ped_positions[:, 0]
    hy = y - past_ped_positions[:, 1]

    mesh = plsc.VectorSubcoreMesh(core_axis_name="c", subcore_axis_name="s")
    sc = functools.partial(
        pl.kernel, mesh=mesh,
        compiler_params=pltpu.CompilerParams(needs_layout_passes=False),
        out_type=[jax.ShapeDtypeStruct((_SC_ROWS,), jnp.float32),
                  jax.ShapeDtypeStruct((_SC_ROWS,), jnp.float32)],
        scratch_types=[
            pltpu.VMEM((n,), jnp.float32),
            pltpu.VMEM((n,), jnp.float32),
            pltpu.VMEM((n,), jnp.float32),
            pltpu.VMEM((n,), jnp.float32),
            pltpu.VMEM((n + _PAD,), jnp.float32),
            pltpu.VMEM((256,), jnp.float32),
            pltpu.VMEM((_NC + 16,), jnp.int32),
            pltpu.VMEM((_NC * 16,), jnp.float32),
            pltpu.VMEM((_RPW,), jnp.float32),
            pltpu.VMEM((_RPW,), jnp.float32),
        ],
    )(_sc_body)
    s15_sc, s16_sc = sc(x, y, hx, hy)

    tc_rows = n - _SC_ROWS
    xrow = x.reshape(1, n)
    yrow = y.reshape(1, n)
    xcol = x[_SC_ROWS:].reshape(tc_rows, 1)
    ycol = y[_SC_ROWS:].reshape(tc_rows, 1)
    hxcol = hx[_SC_ROWS:].reshape(tc_rows, 1)
    hycol = hy[_SC_ROWS:].reshape(tc_rows, 1)

    grid = (tc_rows // _BR,)
    row_spec = pl.BlockSpec((1, n), lambda i: (0, 0))
    col_spec = pl.BlockSpec((_BR, 1), lambda i: (i, 0))
    s15_tc, s16_tc = pl.pallas_call(
        _tc_body,
        grid=grid,
        in_specs=[row_spec, row_spec, col_spec, col_spec, col_spec, col_spec],
        out_specs=[col_spec, col_spec],
        out_shape=[jax.ShapeDtypeStruct((tc_rows, 1), jnp.float32),
                   jax.ShapeDtypeStruct((tc_rows, 1), jnp.float32)],
    )(xrow, yrow, xcol, ycol, hxcol, hycol)

    s15 = jnp.concatenate([s15_sc, s15_tc.reshape(tc_rows)])
    s16 = jnp.concatenate([s16_sc, s16_tc.reshape(tc_rows)])

    shape2 = (32, 128)
    spec = pl.BlockSpec(shape2, lambda: (0, 0))
    out = pl.pallas_call(
        _finish_body,
        in_specs=[spec, spec, spec, spec],
        out_specs=spec,
        out_shape=jax.ShapeDtypeStruct(shape2, jnp.float32),
    )(s15.reshape(shape2), s16.reshape(shape2),
      indexes.astype(jnp.int32).reshape(shape2), all_radii.reshape(shape2))
    return out.reshape(n)


# speedup vs baseline: 1.0953x; 1.0953x over previous
import functools

import jax
import jax.numpy as jnp
from jax import lax
from jax.experimental import pallas as pl
from jax.experimental.pallas import tpu as pltpu
from jax.experimental.pallas import tpu_sc as plsc

_N = 4096
_MAX_PED = 16
_MIN_R = 0.5
_MAX_R = 10.0
_CAP = 400.0
_BIG = 1.0e9
_NW = 32
_SC_ROWS = 512
_RPW = _SC_ROWS // _NW
_NC = 32
_PAD = 256
_BR = 128


def _iota16():
    return lax.iota(jnp.int32, 16)


def _sc_body(x_hbm, y_hbm, hx_hbm, hy_hbm, s15_hbm, s16_hbm,
             xv, yv, hxv, hyv, rowbuf, foldbuf, idbuf, candbuf,
             out15, out16):
    wid = lax.axis_index("s") * 2 + lax.axis_index("c")
    base = wid * _RPW
    pltpu.sync_copy(x_hbm, xv)
    pltpu.sync_copy(y_hbm, yv)
    pltpu.sync_copy(hx_hbm, hxv)
    pltpu.sync_copy(hy_hbm, hyv)
    cap_v = jnp.full((16,), _CAP, jnp.float32)
    for c in range(_PAD // 16):
        rowbuf[pl.ds(_N + c * 16, 16)] = cap_v

    def do_row(iv, _):
        ri = base + iv
        ri_v = _iota16() * 0 + ri
        xi = plsc.load_gather(xv, [ri_v])
        yi = plsc.load_gather(yv, [ri_v])
        hxi = plsc.load_gather(hxv, [ri_v])
        hyi = plsc.load_gather(hyv, [ri_v])
        sqh = hxi * hxi + hyi * hyi

        def do_group(g, _):
            def do_chunk(c8, acc):
                c = g * 16 + c8
                xj = xv[pl.ds(c * 16, 16)]
                yj = yv[pl.ds(c * 16, 16)]
                dx = xj - xi
                dy = yj - yi
                sq = dx * dx + dy * dy
                dt = dx * hxi + dy * hyi
                ok = (dt > 0.0) & (4.0 * (dt * dt) > sq * sqh) & (sq < _CAP)
                val = jnp.where(ok, sq, _CAP)
                rowbuf[pl.ds(c * 16, 16)] = val
                return jnp.minimum(acc, val)

            acc = lax.fori_loop(0, 16, do_chunk, cap_v)
            foldbuf[pl.ds(g * 16, 16)] = acc
            return 0

        lax.fori_loop(0, 16, do_group, 0)

        def t_round(k, m):
            m_v = _iota16() * 0.0 + m

            def scan_g(g, cur):
                fv = foldbuf[pl.ds(g * 16, 16)]
                return jnp.minimum(cur, jnp.where(fv > m_v, fv, _BIG))

            cur = lax.fori_loop(0, 16, scan_g, jnp.full((16,), _BIG, jnp.float32))
            return jnp.min(cur)

        t = lax.fori_loop(0, _MAX_PED + 1, t_round, jnp.float32(-1.0))
        t_v = _iota16() * 0.0 + t

        dummy = jnp.full((16,), 256, jnp.int32)
        for s in range(_NC // 16):
            idbuf[pl.ds(s * 16, 16)] = dummy
        off = jnp.int32(0)
        for g in range(16):
            fv = foldbuf[pl.ds(g * 16, 16)]
            sel = (fv <= t_v) & (fv < _CAP)
            ids = _iota16() + 16 * g
            plsc.store_compressed(idbuf.at[pl.ds(off, 16)], ids, mask=sel)
            off = off + jnp.sum(sel.astype(jnp.int32))

        for s in range(_NC // 16):
            ids = idbuf[pl.ds(s * 16, 16)]
            gq = lax.shift_right_logical(ids, 4)
            lq = lax.bitwise_and(ids, 15)
            for c in range(16):
                cols = 256 * gq + (16 * c) + lq
                candbuf[pl.ds((s * 16 + c) * 16, 16)] = plsc.load_gather(
                    rowbuf, [cols])

        def sel_round(k, carry):
            prev, s15, s16, cum = carry
            prev_v = _iota16() * 0.0 + prev

            def mscan(i, cur):
                v = candbuf[pl.ds(i * 16, 16)]
                return jnp.minimum(cur, jnp.where(v > prev_v, v, _BIG))

            cur = lax.fori_loop(0, _NC, mscan,
                                jnp.full((16,), _BIG, jnp.float32))
            m = jnp.min(cur)
            m_v = _iota16() * 0.0 + m

            def cscan(i, cv):
                v = candbuf[pl.ds(i * 16, 16)]
                return cv + (v == m_v).astype(jnp.int32)

            cnt_v = lax.fori_loop(0, _NC, cscan, jnp.zeros((16,), jnp.int32))
            cnum = jnp.sum(cnt_v)
            nxt = cum + cnum
            s15 = jnp.where((cum <= _MAX_PED - 1) & (nxt > _MAX_PED - 1), m, s15)
            s16 = jnp.where((cum <= _MAX_PED) & (nxt > _MAX_PED), m, s16)
            return m, s15, s16, nxt

        _, s15, s16, _ = lax.fori_loop(
            0, _MAX_PED + 1, sel_round,
            (jnp.float32(-1.0), jnp.float32(_CAP), jnp.float32(_CAP),
             jnp.int32(0)))

        lane0 = _iota16() == 0
        iv_v = _iota16() * 0 + iv
        plsc.store_scatter(out15, [iv_v], _iota16() * 0.0 + s15, mask=lane0)
        plsc.store_scatter(out16, [iv_v], _iota16() * 0.0 + s16, mask=lane0)
        return 0

    lax.fori_loop(0, _RPW, do_row, 0)
    pltpu.sync_copy(out15, s15_hbm.at[pl.ds(base, _RPW)])
    pltpu.sync_copy(out16, s16_hbm.at[pl.ds(base, _RPW)])


def _tc_body(xrow, yrow, xcol, ycol, hxcol, hycol, s15_ref, s16_ref):
    xi = xcol[...]
    yi = ycol[...]
    hx = hxcol[...]
    hy = hycol[...]
    sqh = hx * hx + hy * hy
    xj = xrow[...]
    yj = yrow[...]
    dx = xj - xi
    dy = yj - yi
    squ = dx * dx + dy * dy
    dot = dx * hx + dy * hy
    in_sight = (dot > 0.0) & (4.0 * (dot * dot) > squ * sqh)
    inf = jnp.float32(jnp.inf)
    msk = jnp.where(in_sight, squ, inf)

    ms = []
    m = jnp.full((_BR, 1), -jnp.inf, jnp.float32)
    for _ in range(_MAX_PED + 1):
        m = jnp.min(jnp.where(msk > m, msk, inf), axis=1, keepdims=True)
        ms.append(m)

    cnt17 = jnp.sum((msk <= ms[_MAX_PED]).astype(jnp.int32), axis=1,
                    keepdims=True)
    ties = jnp.any((ms[_MAX_PED] < inf) & (cnt17 > _MAX_PED + 1))

    def fast(_):
        return ms[_MAX_PED - 1], ms[_MAX_PED]

    def slow(_):
        s15 = jnp.full((_BR, 1), inf)
        s16 = jnp.full((_BR, 1), inf)
        cum = jnp.zeros((_BR, 1), jnp.int32)
        for k in range(_MAX_PED + 1):
            c = jnp.sum((msk == ms[k]).astype(jnp.int32), axis=1,
                        keepdims=True)
            nxt = cum + c
            s15 = jnp.where((cum <= _MAX_PED - 1) & (nxt > _MAX_PED - 1),
                            ms[k], s15)
            s16 = jnp.where((cum <= _MAX_PED) & (nxt > _MAX_PED), ms[k], s16)
            cum = nxt
        return s15, s16

    s15, s16 = jax.lax.cond(ties, slow, fast, None)
    s15_ref[...] = s15
    s16_ref[...] = s16


def _finish_body(s15_ref, s16_ref, idx_ref, rad_ref, out_ref):
    r = 0.5 * (jnp.sqrt(s15_ref[...]) + jnp.sqrt(s16_ref[...]))
    r = jnp.clip(r, _MIN_R, _MAX_R)
    out_ref[...] = jnp.where(idx_ref[...] != 0, r, rad_ref[...])


def kernel(past_ped_positions, ped_positions, indexes, all_radii):
    n = ped_positions.shape[0]
    x = ped_positions[:, 0]
    y = ped_positions[:, 1]
    hx = x - past_ped_positions[:, 0]
    hy = y - past_ped_positions[:, 1]

    mesh = plsc.VectorSubcoreMesh(core_axis_name="c", subcore_axis_name="s")
    sc = functools.partial(
        pl.kernel, mesh=mesh,
        compiler_params=pltpu.CompilerParams(needs_layout_passes=False),
        out_type=[jax.ShapeDtypeStruct((_SC_ROWS,), jnp.float32),
                  jax.ShapeDtypeStruct((_SC_ROWS,), jnp.float32)],
        scratch_types=[
            pltpu.VMEM((n,), jnp.float32),
            pltpu.VMEM((n,), jnp.float32),
            pltpu.VMEM((n,), jnp.float32),
            pltpu.VMEM((n,), jnp.float32),
            pltpu.VMEM((n + _PAD,), jnp.float32),
            pltpu.VMEM((256,), jnp.float32),
            pltpu.VMEM((_NC + 16,), jnp.int32),
            pltpu.VMEM((_NC * 16,), jnp.float32),
            pltpu.VMEM((_RPW,), jnp.float32),
            pltpu.VMEM((_RPW,), jnp.float32),
        ],
    )(_sc_body)
    s15_sc, s16_sc = sc(x, y, hx, hy)

    tc_rows = n - _SC_ROWS
    xrow = x.reshape(1, n)
    yrow = y.reshape(1, n)
    xcol = x[_SC_ROWS:].reshape(tc_rows, 1)
    ycol = y[_SC_ROWS:].reshape(tc_rows, 1)
    hxcol = hx[_SC_ROWS:].reshape(tc_rows, 1)
    hycol = hy[_SC_ROWS:].reshape(tc_rows, 1)

    grid = (tc_rows // _BR,)
    row_spec = pl.BlockSpec((1, n), lambda i: (0, 0))
    col_spec = pl.BlockSpec((_BR, 1), lambda i: (i, 0))
    s15_tc, s16_tc = pl.pallas_call(
        _tc_body,
        grid=grid,
        in_specs=[row_spec, row_spec, col_spec, col_spec, col_spec, col_spec],
        out_specs=[col_spec, col_spec],
        out_shape=[jax.ShapeDtypeStruct((tc_rows, 1), jnp.float32),
                   jax.ShapeDtypeStruct((tc_rows, 1), jnp.float32)],
    )(xrow, yrow, xcol, ycol, hxcol, hycol)

    s15 = jnp.concatenate([s15_sc, s15_tc.reshape(tc_rows)])
    s16 = jnp.concatenate([s16_sc, s16_tc.reshape(tc_rows)])

    shape2 = (32, 128)
    spec = pl.BlockSpec(shape2, lambda: (0, 0))
    out = pl.pallas_call(
        _finish_body,
        in_specs=[spec, spec, spec, spec],
        out_specs=spec,
        out_shape=jax.ShapeDtypeStruct(shape2, jnp.float32),
    )(s15.reshape(shape2), s16.reshape(shape2),
      indexes.astype(jnp.int32).reshape(shape2), all_radii.reshape(shape2))
    return out.reshape(n)


# speedup vs baseline: 1.1450x; 1.0454x over previous
import functools

import jax
import jax.numpy as jnp
from jax import lax
from jax.experimental import pallas as pl
from jax.experimental.pallas import tpu as pltpu
from jax.experimental.pallas import tpu_sc as plsc

_N = 4096
_MAX_PED = 16
_MIN_R = 0.5
_MAX_R = 10.0
_CAP = 400.0
_BIG = 1.0e9
_NW = 32
_SC_ROWS = 512
_RPW = _SC_ROWS // _NW
_NC = 32
_PAD = 256
_BR = 256


def _iota16():
    return lax.iota(jnp.int32, 16)


def _sc_body(x_hbm, y_hbm, hx_hbm, hy_hbm, s15_hbm, s16_hbm,
             xv, yv, hxv, hyv, rowbuf, foldbuf, idbuf, candbuf,
             out15, out16):
    wid = lax.axis_index("s") * 2 + lax.axis_index("c")
    base = wid * _RPW
    pltpu.sync_copy(x_hbm, xv)
    pltpu.sync_copy(y_hbm, yv)
    pltpu.sync_copy(hx_hbm, hxv)
    pltpu.sync_copy(hy_hbm, hyv)
    cap_v = jnp.full((16,), _CAP, jnp.float32)
    for c in range(_PAD // 16):
        rowbuf[pl.ds(_N + c * 16, 16)] = cap_v

    def do_row(iv, _):
        ri = base + iv
        ri_v = _iota16() * 0 + ri
        xi = plsc.load_gather(xv, [ri_v])
        yi = plsc.load_gather(yv, [ri_v])
        hxi = plsc.load_gather(hxv, [ri_v])
        hyi = plsc.load_gather(hyv, [ri_v])
        sqh = hxi * hxi + hyi * hyi

        def do_group(g, _):
            def do_chunk(c8, acc):
                c = g * 16 + c8
                xj = xv[pl.ds(c * 16, 16)]
                yj = yv[pl.ds(c * 16, 16)]
                dx = xj - xi
                dy = yj - yi
                sq = dx * dx + dy * dy
                dt = dx * hxi + dy * hyi
                ok = (dt > 0.0) & (4.0 * (dt * dt) > sq * sqh) & (sq < _CAP)
                val = jnp.where(ok, sq, _CAP)
                rowbuf[pl.ds(c * 16, 16)] = val
                return jnp.minimum(acc, val)

            acc = lax.fori_loop(0, 16, do_chunk, cap_v)
            foldbuf[pl.ds(g * 16, 16)] = acc
            return 0

        lax.fori_loop(0, 16, do_group, 0)

        def t_round(k, m):
            m_v = _iota16() * 0.0 + m

            def scan_g(g, cur):
                fv = foldbuf[pl.ds(g * 16, 16)]
                return jnp.minimum(cur, jnp.where(fv > m_v, fv, _BIG))

            cur = lax.fori_loop(0, 16, scan_g, jnp.full((16,), _BIG, jnp.float32))
            return jnp.min(cur)

        t = lax.fori_loop(0, _MAX_PED + 1, t_round, jnp.float32(-1.0))
        t_v = _iota16() * 0.0 + t

        dummy = jnp.full((16,), 256, jnp.int32)
        for s in range(_NC // 16):
            idbuf[pl.ds(s * 16, 16)] = dummy
        off = jnp.int32(0)
        for g in range(16):
            fv = foldbuf[pl.ds(g * 16, 16)]
            sel = (fv <= t_v) & (fv < _CAP)
            ids = _iota16() + 16 * g
            plsc.store_compressed(idbuf.at[pl.ds(off, 16)], ids, mask=sel)
            off = off + jnp.sum(sel.astype(jnp.int32))

        for s in range(_NC // 16):
            ids = idbuf[pl.ds(s * 16, 16)]
            gq = lax.shift_right_logical(ids, 4)
            lq = lax.bitwise_and(ids, 15)
            for c in range(16):
                cols = 256 * gq + (16 * c) + lq
                candbuf[pl.ds((s * 16 + c) * 16, 16)] = plsc.load_gather(
                    rowbuf, [cols])

        def sel_round(k, carry):
            prev, s15, s16, cum = carry
            prev_v = _iota16() * 0.0 + prev

            def mscan(i, cur):
                v = candbuf[pl.ds(i * 16, 16)]
                return jnp.minimum(cur, jnp.where(v > prev_v, v, _BIG))

            cur = lax.fori_loop(0, _NC, mscan,
                                jnp.full((16,), _BIG, jnp.float32))
            m = jnp.min(cur)
            m_v = _iota16() * 0.0 + m

            def cscan(i, cv):
                v = candbuf[pl.ds(i * 16, 16)]
                return cv + (v == m_v).astype(jnp.int32)

            cnt_v = lax.fori_loop(0, _NC, cscan, jnp.zeros((16,), jnp.int32))
            cnum = jnp.sum(cnt_v)
            nxt = cum + cnum
            s15 = jnp.where((cum <= _MAX_PED - 1) & (nxt > _MAX_PED - 1), m, s15)
            s16 = jnp.where((cum <= _MAX_PED) & (nxt > _MAX_PED), m, s16)
            return m, s15, s16, nxt

        _, s15, s16, _ = lax.fori_loop(
            0, _MAX_PED + 1, sel_round,
            (jnp.float32(-1.0), jnp.float32(_CAP), jnp.float32(_CAP),
             jnp.int32(0)))

        lane0 = _iota16() == 0
        iv_v = _iota16() * 0 + iv
        plsc.store_scatter(out15, [iv_v], _iota16() * 0.0 + s15, mask=lane0)
        plsc.store_scatter(out16, [iv_v], _iota16() * 0.0 + s16, mask=lane0)
        return 0

    lax.fori_loop(0, _RPW, do_row, 0)
    pltpu.sync_copy(out15, s15_hbm.at[pl.ds(base, _RPW)])
    pltpu.sync_copy(out16, s16_hbm.at[pl.ds(base, _RPW)])


def _tc_body(xrow, yrow, xcol, ycol, hxcol, hycol, s15_ref, s16_ref):
    xi = xcol[...]
    yi = ycol[...]
    hx = hxcol[...]
    hy = hycol[...]
    sqh = hx * hx + hy * hy
    xj = xrow[...]
    yj = yrow[...]
    dx = xj - xi
    dy = yj - yi
    squ = dx * dx + dy * dy
    dot = dx * hx + dy * hy
    in_sight = (dot > 0.0) & (4.0 * (dot * dot) > squ * sqh)
    inf = jnp.float32(jnp.inf)
    msk = jnp.where(in_sight, squ, inf)

    ms = []
    m = jnp.full((_BR, 1), -jnp.inf, jnp.float32)
    for _ in range(_MAX_PED + 1):
        m = jnp.min(jnp.where(msk > m, msk, inf), axis=1, keepdims=True)
        ms.append(m)

    cnt17 = jnp.sum((msk <= ms[_MAX_PED]).astype(jnp.int32), axis=1,
                    keepdims=True)
    ties = jnp.any((ms[_MAX_PED] < inf) & (cnt17 > _MAX_PED + 1))

    def fast(_):
        return ms[_MAX_PED - 1], ms[_MAX_PED]

    def slow(_):
        s15 = jnp.full((_BR, 1), inf)
        s16 = jnp.full((_BR, 1), inf)
        cum = jnp.zeros((_BR, 1), jnp.int32)
        for k in range(_MAX_PED + 1):
            c = jnp.sum((msk == ms[k]).astype(jnp.int32), axis=1,
                        keepdims=True)
            nxt = cum + c
            s15 = jnp.where((cum <= _MAX_PED - 1) & (nxt > _MAX_PED - 1),
                            ms[k], s15)
            s16 = jnp.where((cum <= _MAX_PED) & (nxt > _MAX_PED), ms[k], s16)
            cum = nxt
        return s15, s16

    s15, s16 = jax.lax.cond(ties, slow, fast, None)
    s15_ref[...] = s15
    s16_ref[...] = s16


def _finish_body(s15_ref, s16_ref, idx_ref, rad_ref, out_ref):
    r = 0.5 * (jnp.sqrt(s15_ref[...]) + jnp.sqrt(s16_ref[...]))
    r = jnp.clip(r, _MIN_R, _MAX_R)
    out_ref[...] = jnp.where(idx_ref[...] != 0, r, rad_ref[...])


def kernel(past_ped_positions, ped_positions, indexes, all_radii):
    n = ped_positions.shape[0]
    x = ped_positions[:, 0]
    y = ped_positions[:, 1]
    hx = x - past_ped_positions[:, 0]
    hy = y - past_ped_positions[:, 1]

    mesh = plsc.VectorSubcoreMesh(core_axis_name="c", subcore_axis_name="s")
    sc = functools.partial(
        pl.kernel, mesh=mesh,
        compiler_params=pltpu.CompilerParams(needs_layout_passes=False),
        out_type=[jax.ShapeDtypeStruct((_SC_ROWS,), jnp.float32),
                  jax.ShapeDtypeStruct((_SC_ROWS,), jnp.float32)],
        scratch_types=[
            pltpu.VMEM((n,), jnp.float32),
            pltpu.VMEM((n,), jnp.float32),
            pltpu.VMEM((n,), jnp.float32),
            pltpu.VMEM((n,), jnp.float32),
            pltpu.VMEM((n + _PAD,), jnp.float32),
            pltpu.VMEM((256,), jnp.float32),
            pltpu.VMEM((_NC + 16,), jnp.int32),
            pltpu.VMEM((_NC * 16,), jnp.float32),
            pltpu.VMEM((_RPW,), jnp.float32),
            pltpu.VMEM((_RPW,), jnp.float32),
        ],
    )(_sc_body)
    s15_sc, s16_sc = sc(x, y, hx, hy)

    tc_rows = n - _SC_ROWS
    xrow = x.reshape(1, n)
    yrow = y.reshape(1, n)
    xcol = x[_SC_ROWS:].reshape(tc_rows, 1)
    ycol = y[_SC_ROWS:].reshape(tc_rows, 1)
    hxcol = hx[_SC_ROWS:].reshape(tc_rows, 1)
    hycol = hy[_SC_ROWS:].reshape(tc_rows, 1)

    grid = (tc_rows // _BR,)
    row_spec = pl.BlockSpec((1, n), lambda i: (0, 0))
    col_spec = pl.BlockSpec((_BR, 1), lambda i: (i, 0))
    s15_tc, s16_tc = pl.pallas_call(
        _tc_body,
        grid=grid,
        in_specs=[row_spec, row_spec, col_spec, col_spec, col_spec, col_spec],
        out_specs=[col_spec, col_spec],
        out_shape=[jax.ShapeDtypeStruct((tc_rows, 1), jnp.float32),
                   jax.ShapeDtypeStruct((tc_rows, 1), jnp.float32)],
    )(xrow, yrow, xcol, ycol, hxcol, hycol)

    s15 = jnp.concatenate([s15_sc, s15_tc.reshape(tc_rows)])
    s16 = jnp.concatenate([s16_sc, s16_tc.reshape(tc_rows)])

    shape2 = (32, 128)
    spec = pl.BlockSpec(shape2, lambda: (0, 0))
    out = pl.pallas_call(
        _finish_body,
        in_specs=[spec, spec, spec, spec],
        out_specs=spec,
        out_shape=jax.ShapeDtypeStruct(shape2, jnp.float32),
    )(s15.reshape(shape2), s16.reshape(shape2),
      indexes.astype(jnp.int32).reshape(shape2), all_radii.reshape(shape2))
    return out.reshape(n)
